# baseline (speedup 1.0000x reference)
import functools

import jax
import jax.numpy as jnp
from jax import lax
from jax.experimental import pallas as pl
from jax.experimental.pallas import tpu as pltpu
from jax.experimental.pallas import tpu_sc as plsc

DIMS = 64
L = 16
NC = 2
NS = 16
NW = NC * NS

EPS2 = 1e-24


def _make_sc_kernel(B):
    b_per_w = B // NW
    CH = 128
    n_ch = b_per_w // CH
    n_grp = CH // L

    mesh = plsc.VectorSubcoreMesh(
        core_axis_name="c", subcore_axis_name="s",
        num_cores=NC, num_subcores=NS)

    @functools.partial(
        pl.kernel,
        out_type=(jax.ShapeDtypeStruct((B,), jnp.float32),
                  jax.ShapeDtypeStruct((B,), jnp.float32)),
        mesh=mesh,
        scratch_types=[
            pltpu.VMEM((b_per_w, 5), jnp.int32),
            pltpu.VMEM((b_per_w,), jnp.int32),
            pltpu.VMEM((b_per_w,), jnp.int32),
            pltpu.VMEM((b_per_w,), jnp.int32),
            pltpu.VMEM((b_per_w,), jnp.int32),
            pltpu.VMEM((b_per_w,), jnp.int32),
            [[pltpu.VMEM((DIMS, CH), jnp.float32)
              for _ in range(6)] for _ in range(2)],
            pltpu.VMEM((b_per_w,), jnp.float32),
            pltpu.VMEM((b_per_w,), jnp.float32),
            [pltpu.SemaphoreType.DMA for _ in range(2)],
        ],
        compiler_params=pltpu.CompilerParams(
            needs_layout_passes=False, use_tc_tiling_on_sc=False),
    )
    def transh(data_hbm, ent_hbm, rel_hbm, wrel_hbm,
               dpos_hbm, dneg_hbm,
               raw_v, hi_v, ti_v, ri_v, chi_v, cti_v,
               bufs, po_v, ne_v, sems):
        wid = lax.axis_index("s") * NC + lax.axis_index("c")
        base = wid * b_per_w

        pltpu.sync_copy(data_hbm.at[pl.ds(base, b_per_w)], raw_v)

        lane_ids = lax.iota(jnp.int32, L)

        def split_body(g, _):
            rows = g * L + lane_ids
            for col, dst in ((0, hi_v), (1, ti_v), (2, ri_v),
                             (3, chi_v), (4, cti_v)):
                vals = plsc.load_gather(
                    raw_v, [rows, jnp.full((L,), col, jnp.int32)])
                dst[pl.ds(g * L, L)] = vals
            return _

        lax.fori_loop(0, b_per_w // L, split_body, None)

        def roles(k):
            hv, tv, rv, wv, chv, ctv = bufs[k]
            return ((ent_hbm, hi_v, hv), (ent_hbm, ti_v, tv),
                    (rel_hbm, ri_v, rv), (wrel_hbm, ri_v, wv),
                    (ent_hbm, chi_v, chv), (ent_hbm, cti_v, ctv))

        def fire(c, k):
            sl = pl.ds(c * CH, CH)
            for tbl, idx, buf in roles(k):
                def dim_body(d, _, tbl=tbl, idx=idx, buf=buf):
                    pltpu.async_copy(tbl.at[d].at[idx.at[sl]],
                                     buf.at[d], sems[k])
                    return _
                lax.fori_loop(0, DIMS, dim_body, None)

        def drain(k):
            for tbl, _idx, buf in roles(k):
                pltpu.make_async_copy(tbl.at[0].at[pl.ds(0, CH)],
                                      buf, sems[k]).wait()

        fire(0, 0)
        for c in range(n_ch):
            k = c % 2
            if c + 1 < n_ch:
                fire(c + 1, (c + 1) % 2)
            drain(k)
            hv, tv, rv, wv, chv, ctv = bufs[k]

            def grp_body(g, _, hv=hv, tv=tv, rv=rv, wv=wv, chv=chv,
                         ctv=ctv, c=c):
                zeros = jnp.zeros((L,), jnp.float32)
                U = V = A = C = R = S = zeros
                for d in range(DIMS):
                    sl16 = pl.ds(g * L, L)
                    hl = hv[d, sl16]
                    tl = tv[d, sl16]
                    rl = rv[d, sl16]
                    wl = wv[d, sl16]
                    chl = chv[d, sl16]
                    ctl = ctv[d, sl16]
                    hmt = hl - tl
                    cc = chl - ctl
                    u = hmt + rl
                    v = cc + rl
                    U = U + u * u
                    V = V + v * v
                    A = A + hmt * wl
                    C = C + cc * wl
                    R = R + rl * wl
                    S = S + wl * wl
                invS = 1.0 / jnp.maximum(S, EPS2)
                po_v[pl.ds(c * CH + g * L, L)] = (A * A + 2.0 * A * R) * invS - U
                ne_v[pl.ds(c * CH + g * L, L)] = (C * C + 2.0 * C * R) * invS - V
                return _

            lax.fori_loop(0, n_grp, grp_body, None)

        pltpu.sync_copy(po_v, dpos_hbm.at[pl.ds(base, b_per_w)])
        pltpu.sync_copy(ne_v, dneg_hbm.at[pl.ds(base, b_per_w)])

    return transh


@jax.jit
def kernel(data, entities, relations, w_relations):
    B = data.shape[0]
    transh = _make_sc_kernel(B)
    d_pos, d_neg = transh(data, jnp.transpose(entities),
                          jnp.transpose(relations),
                          jnp.transpose(w_relations))
    t_const = jnp.full((B, 1), -1.0, jnp.float32)
    return (d_pos, d_neg, t_const)


# speedup vs baseline: 9.0126x; 9.0126x over previous
import functools

import jax
import jax.numpy as jnp
from jax import lax
from jax.experimental import pallas as pl
from jax.experimental.pallas import tpu as pltpu
from jax.experimental.pallas import tpu_sc as plsc

DIMS = 64
L = 16
NC = 2
NS = 16
NW = NC * NS
KD = DIMS // L

EPS2 = 1e-24


def _make_sc_kernel(B):
    b_per_w = B // NW
    CH = 128
    n_ch = b_per_w // CH
    n_grp = CH // L

    mesh = plsc.VectorSubcoreMesh(
        core_axis_name="c", subcore_axis_name="s",
        num_cores=NC, num_subcores=NS)

    @functools.partial(
        pl.kernel,
        out_type=(jax.ShapeDtypeStruct((B,), jnp.float32),
                  jax.ShapeDtypeStruct((B,), jnp.float32)),
        mesh=mesh,
        scratch_types=[
            pltpu.VMEM((b_per_w, 5), jnp.int32),
            pltpu.VMEM((b_per_w,), jnp.int32),
            pltpu.VMEM((b_per_w,), jnp.int32),
            pltpu.VMEM((b_per_w,), jnp.int32),
            pltpu.VMEM((b_per_w,), jnp.int32),
            pltpu.VMEM((b_per_w,), jnp.int32),
            [[pltpu.VMEM((CH, DIMS), jnp.float32)
              for _ in range(6)] for _ in range(2)],
            pltpu.VMEM((b_per_w,), jnp.float32),
            pltpu.VMEM((b_per_w,), jnp.float32),
            [pltpu.SemaphoreType.DMA for _ in range(2)],
        ],
        compiler_params=pltpu.CompilerParams(
            needs_layout_passes=False, use_tc_tiling_on_sc=False),
    )
    def transh(data_hbm, ent_hbm, rel_hbm, wrel_hbm,
               dpos_hbm, dneg_hbm,
               raw_v, hi_v, ti_v, ri_v, chi_v, cti_v,
               bufs, po_v, ne_v, sems):
        wid = lax.axis_index("s") * NC + lax.axis_index("c")
        base = wid * b_per_w

        pltpu.sync_copy(data_hbm.at[pl.ds(base, b_per_w)], raw_v)

        lane_ids = lax.iota(jnp.int32, L)

        def split_body(g, _):
            rows = g * L + lane_ids
            for col, dst in ((0, hi_v), (1, ti_v), (2, ri_v),
                             (3, chi_v), (4, cti_v)):
                vals = plsc.load_gather(
                    raw_v, [rows, jnp.full((L,), col, jnp.int32)])
                dst[pl.ds(g * L, L)] = vals
            return _

        lax.fori_loop(0, b_per_w // L, split_body, None)

        def fire(c, k):
            sl = pl.ds(c * CH, CH)
            hv, tv, rv, wv, chv, ctv = bufs[k]
            return [
                pltpu.async_copy(ent_hbm.at[hi_v.at[sl]], hv, sems[k]),
                pltpu.async_copy(ent_hbm.at[ti_v.at[sl]], tv, sems[k]),
                pltpu.async_copy(rel_hbm.at[ri_v.at[sl]], rv, sems[k]),
                pltpu.async_copy(wrel_hbm.at[ri_v.at[sl]], wv, sems[k]),
                pltpu.async_copy(ent_hbm.at[chi_v.at[sl]], chv, sems[k]),
                pltpu.async_copy(ent_hbm.at[cti_v.at[sl]], ctv, sems[k]),
            ]

        pending = {0: fire(0, 0)}
        for c in range(n_ch):
            k = c % 2
            if c + 1 < n_ch:
                pending[c + 1] = fire(c + 1, (c + 1) % 2)
            for cp in pending.pop(c):
                cp.wait()
            hv, tv, rv, wv, chv, ctv = bufs[k]

            def grp_body(g, _, hv=hv, tv=tv, rv=rv, wv=wv, chv=chv,
                         ctv=ctv, c=c):
                row_ids = g * L + lane_ids
                zeros = jnp.zeros((L,), jnp.float32)
                U = V = A = C = R = S = zeros
                for d in range(DIMS):
                    cols = jnp.full((L,), d, jnp.int32)
                    hl = plsc.load_gather(hv, [row_ids, cols])
                    tl = plsc.load_gather(tv, [row_ids, cols])
                    rl = plsc.load_gather(rv, [row_ids, cols])
                    wl = plsc.load_gather(wv, [row_ids, cols])
                    chl = plsc.load_gather(chv, [row_ids, cols])
                    ctl = plsc.load_gather(ctv, [row_ids, cols])
                    hmt = hl - tl
                    cc = chl - ctl
                    u = hmt + rl
                    v = cc + rl
                    U = U + u * u
                    V = V + v * v
                    A = A + hmt * wl
                    C = C + cc * wl
                    R = R + rl * wl
                    S = S + wl * wl
                invS = 1.0 / jnp.maximum(S, EPS2)
                po_v[pl.ds(c * CH + g * L, L)] = (A * A + 2.0 * A * R) * invS - U
                ne_v[pl.ds(c * CH + g * L, L)] = (C * C + 2.0 * C * R) * invS - V
                return _

            lax.fori_loop(0, n_grp, grp_body, None)

        pltpu.sync_copy(po_v, dpos_hbm.at[pl.ds(base, b_per_w)])
        pltpu.sync_copy(ne_v, dneg_hbm.at[pl.ds(base, b_per_w)])

    return transh


@jax.jit
def kernel(data, entities, relations, w_relations):
    B = data.shape[0]
    transh = _make_sc_kernel(B)
    d_pos, d_neg = transh(data, entities, relations, w_relations)
    t_const = jnp.full((B, 1), -1.0, jnp.float32)
    return (d_pos, d_neg, t_const)
